# Initial kernel scaffold; baseline (speedup 1.0000x reference)
import jax
import jax.numpy as jnp
from jax.experimental import pallas as pl


def kernel(x, table, W, b):
    raise NotImplementedError("write your pallas kernel here")


# speedup vs baseline: 2.5216x; 2.5216x over previous
import functools

import jax
import jax.numpy as jnp
from jax import lax
from jax.experimental import pallas as pl
from jax.experimental.pallas import tpu as pltpu
from jax.experimental.pallas import tpu_sc as plsc

VOCAB = 1000000
EMBED_DIM = 32
BATCH = 16384
HIST = 50
N_IDX = BATCH * HIST
NC, NS = 2, 16
NW = NC * NS
PER_TILE = N_IDX // NW
ROWS = 128
N_GATHER = PER_TILE // ROWS
N_PAIR = N_GATHER // 2

_mesh = plsc.VectorSubcoreMesh(
    core_axis_name="c", subcore_axis_name="s", num_cores=NC, num_subcores=NS
)


@functools.partial(
    pl.kernel,
    out_type=jax.ShapeDtypeStruct((NW, 16), jnp.float32),
    mesh=_mesh,
    compiler_params=pltpu.CompilerParams(use_tc_tiling_on_sc=False),
    scratch_types=[
        pltpu.VMEM((N_GATHER, ROWS), jnp.int32),
        pltpu.VMEM((ROWS, EMBED_DIM), jnp.float32),
        pltpu.VMEM((ROWS, EMBED_DIM), jnp.float32),
        pltpu.VMEM((EMBED_DIM,), jnp.float32),
        pltpu.VMEM((16,), jnp.float32),
        pltpu.SemaphoreType.DMA,
        pltpu.SemaphoreType.DMA,
    ],
)
def _gather_reduce(idx_hbm, table_hbm, w_hbm, out_hbm,
                   idx_v, buf0, buf1, w_v, out_v, sem0, sem1):
    wid = lax.axis_index("s") * NC + lax.axis_index("c")
    pltpu.sync_copy(idx_hbm.at[wid], idx_v)
    pltpu.sync_copy(w_hbm, w_v)

    def accum(buf, acc):
        def body(r, carry):
            a0, a1 = carry
            return (a0 + buf[r, pl.ds(0, 16)], a1 + buf[r, pl.ds(16, 16)])
        return lax.fori_loop(0, ROWS, body, acc, unroll=8)

    def wait(buf, sem):
        pltpu.make_async_copy(table_hbm.at[idx_v.at[0]], buf, sem).wait()

    pltpu.async_copy(table_hbm.at[idx_v.at[0]], buf0, sem0)

    zero = jnp.zeros((16,), jnp.float32)

    def pair(t, acc):
        g0 = 2 * t
        wait(buf0, sem0)
        pltpu.async_copy(table_hbm.at[idx_v.at[g0 + 1]], buf1, sem1)
        acc = accum(buf0, acc)
        wait(buf1, sem1)
        g2 = lax.rem(g0 + 2, N_GATHER)
        pltpu.async_copy(table_hbm.at[idx_v.at[g2]], buf0, sem0)
        return accum(buf1, acc)

    a0, a1 = lax.fori_loop(0, N_PAIR, pair, (zero, zero))
    wait(buf0, sem0)

    out_v[...] = a0 * w_v[pl.ds(0, 16)] + a1 * w_v[pl.ds(16, 16)]
    pltpu.sync_copy(out_v, out_hbm.at[wid])


def kernel(x, table, W, b):
    idx = x.reshape(NW, N_GATHER, ROWS)
    partials = _gather_reduce(idx, table, W.reshape(EMBED_DIM))
    return jnp.sum(partials) / jnp.float32(N_IDX) + b[0]


# speedup vs baseline: 2.8232x; 1.1196x over previous
import functools

import jax
import jax.numpy as jnp
from jax import lax
from jax.experimental import pallas as pl
from jax.experimental.pallas import tpu as pltpu
from jax.experimental.pallas import tpu_sc as plsc

VOCAB = 1000000
EMBED_DIM = 32
BATCH = 16384
HIST = 50
N_IDX = BATCH * HIST
NC, NS = 2, 16
NW = NC * NS
PER_TILE = N_IDX // NW
ROWS = 128
N_GATHER = PER_TILE // ROWS
N_PAIR = N_GATHER // 2

_mesh = plsc.VectorSubcoreMesh(
    core_axis_name="c", subcore_axis_name="s", num_cores=NC, num_subcores=NS
)


@functools.partial(
    pl.kernel,
    out_type=jax.ShapeDtypeStruct((NW, 16), jnp.float32),
    mesh=_mesh,
    compiler_params=pltpu.CompilerParams(use_tc_tiling_on_sc=False),
    scratch_types=[
        pltpu.VMEM((N_GATHER, ROWS), jnp.int32),
        pltpu.VMEM((ROWS, EMBED_DIM), jnp.float32),
        pltpu.VMEM((ROWS, EMBED_DIM), jnp.float32),
        pltpu.VMEM((EMBED_DIM,), jnp.float32),
        pltpu.VMEM((16,), jnp.float32),
        pltpu.SemaphoreType.DMA,
        pltpu.SemaphoreType.DMA,
    ],
)
def _gather_reduce(idx_hbm, table_hbm, w_hbm, out_hbm,
                   idx_v, buf0, buf1, w_v, out_v, sem0, sem1):
    wid = lax.axis_index("s") * NC + lax.axis_index("c")
    pltpu.sync_copy(idx_hbm.at[wid], idx_v)
    pltpu.sync_copy(w_hbm, w_v)

    zero = jnp.zeros((16,), jnp.float32)

    def clear(r, _):
        buf0[r, pl.ds(0, 16)] = zero
        buf0[r, pl.ds(16, 16)] = zero
        buf1[r, pl.ds(0, 16)] = zero
        buf1[r, pl.ds(16, 16)] = zero
        return 0

    lax.fori_loop(0, ROWS, clear, 0)

    def wait(buf, sem):
        pltpu.make_async_copy(table_hbm.at[idx_v.at[0]], buf, sem).wait()

    pltpu.async_copy(table_hbm.at[idx_v.at[0]], buf0, sem0, add=True)
    pltpu.async_copy(table_hbm.at[idx_v.at[1]], buf1, sem1, add=True)

    def pair(t, carry):
        g = 2 * t + 2
        wait(buf0, sem0)
        pltpu.async_copy(table_hbm.at[idx_v.at[g]], buf0, sem0, add=True)
        wait(buf1, sem1)
        pltpu.async_copy(table_hbm.at[idx_v.at[g + 1]], buf1, sem1, add=True)
        return carry

    lax.fori_loop(0, N_PAIR - 1, pair, 0)
    wait(buf0, sem0)
    wait(buf1, sem1)

    def accum(buf, acc):
        def body(i, carry):
            r = 4 * i
            c = list(carry)
            for j in range(4):
                c[2 * j] = c[2 * j] + buf[r + j, pl.ds(0, 16)]
                c[2 * j + 1] = c[2 * j + 1] + buf[r + j, pl.ds(16, 16)]
            return tuple(c)
        return lax.fori_loop(0, ROWS // 4, body, acc, unroll=2)

    acc = (zero,) * 8
    acc = accum(buf0, acc)
    acc = accum(buf1, acc)
    a0 = acc[0] + acc[2] + acc[4] + acc[6]
    a1 = acc[1] + acc[3] + acc[5] + acc[7]

    out_v[...] = a0 * w_v[pl.ds(0, 16)] + a1 * w_v[pl.ds(16, 16)]
    pltpu.sync_copy(out_v, out_hbm.at[wid])


def kernel(x, table, W, b):
    idx = x.reshape(NW, N_GATHER, ROWS)
    partials = _gather_reduce(idx, table, W.reshape(EMBED_DIM))
    return jnp.sum(partials) / jnp.float32(N_IDX) + b[0]
